# baseline (speedup 1.0000x reference)
import jax
import jax.numpy as jnp
from jax.experimental import pallas as pl
from jax.experimental.pallas import tpu as pltpu

A, NN, F, RES, NI = 128, 48, 128, 20, 3
CH = 32
C = A // CH
CHN = CH * NN
CUTOFF = 5.0
EPS = 1e-8
_F32 = jnp.float32
_C9, _C10, _C11 = 55.0, 99.0, 45.0
_SQ2C = (2.0 / CUTOFF) ** 0.5


def _sig(x):
    return 1.0 / (1.0 + jnp.exp(-x))


def _swish(x):
    return x * _sig(x)


def _swishp(z):
    s = _sig(z)
    return s * (1.0 + z * (1.0 - s))


_BF16 = jnp.bfloat16


def _split(x):
    xh = x.astype(_BF16)
    xl = (x - xh.astype(_F32)).astype(_BF16)
    return xh, xl


def _rawdot(x, y):
    return jnp.dot(x, y, preferred_element_type=_F32)


def _dot(x, y):
    xh, xl = _split(x)
    yh, yl = _split(y)
    return _rawdot(xh, yh) + (_rawdot(xh, yl) + _rawdot(xl, yh))


def _dot1(oh, y):
    ohb = oh.astype(_BF16)
    yh, yl = _split(y)
    return _rawdot(ohb, yh) + _rawdot(ohb, yl)


def _iota(shape, dim):
    return jax.lax.broadcasted_iota(jnp.int32, shape, dim)


def _mlp2(x, w0, b0, w1, b1):
    h = _dot(x, w0)
    if b0 is not None:
        h = h + b0
    y = _dot(_swish(h), w1)
    return y + b1 if b1 is not None else y


def _mlp2_bwd(dy, x, w0, b0, w0t, w1t):
    z0 = _dot(x, w0)
    if b0 is not None:
        z0 = z0 + b0
    return _dot(_dot(dy, w1t) * _swishp(z0), w0t)


def _body(r_ref, z_ref, ncol_ref, nrow_ref, am_ref, nm_ref,
          wff_ref, wfft_ref, bff_ref, rbfw_ref, rbfwt_ref, rbfb_ref,
          fw_ref, emb_ref, at0_ref, at0b_ref, at1_ref, at1b_ref,
          at2_ref, at2b_ref, at0t_ref, at1t_ref,
          e_ref, ff_ref, fd_ref,
          s_a, s_r, s_f, s_am, s_t3, s_drx, s_dam, s_drg, s_dpr,
          s_fdir, s_dr, s_ohc, s_ohtc, s_rbfc, s_gsc):
    rfull = r_ref[0]
    zb = z_ref[0]
    emb = emb_ref[...]

    ai_col = _iota((CHN, 1), 0) // NN
    rep = (_iota((CHN, CH), 1) == ai_col).astype(_BF16)
    rept = (_iota((CH, CHN), 0) == (_iota((1, CHN), 1) // NN)).astype(_BF16)
    nv = ((_iota((1, RES), 1) + 1).astype(_F32) * (jnp.pi / CUTOFF))

    def lw(k):
        return wff_ref[k]

    def lwt(k):
        return wfft_ref[k]

    def lb(k):
        return bff_ref[pl.ds(k, 1), :]

    def geom_init(cc, tok):
        rowsn = pl.ds(cc * CHN, CHN)
        ncol = ncol_ref[0, rowsn, :]
        nrow = nrow_ref[0, cc]
        oh = (_iota((CHN, A), 1) == ncol).astype(_BF16)
        oht = (_iota((A, CHN), 0) == nrow).astype(_BF16)
        s_ohc[rowsn, :] = oh
        s_ohtc[:, rowsn] = oht
        rg = _dot1(oh, rfull)
        ri = _dot1(rep, r_ref[0, pl.ds(cc * CH, CH), :])
        vec = rg - ri
        v0, v1, v2 = vec[:, 0:1], vec[:, 1:2], vec[:, 2:3]
        d = jnp.sqrt(v0 * v0 + v1 * v1 + v2 * v2 + EPS)
        dx = d + EPS
        invdx = 1.0 / dx
        sn = jnp.sin(nv * d)
        cn = jnp.cos(nv * d)
        rbf = _SQ2C * sn * invdx
        drbf = _SQ2C * (nv * cn * dx - sn) * (invdx * invdx)
        x = d * (1.0 / CUTOFF)
        x2 = x * x
        x4 = x2 * x2
        x8 = x4 * x4
        lt1 = x < 1.0
        cut = jnp.where(lt1, 1.0 - x8 * x * (_C9 - _C10 * x + _C11 * x2),
                        0.0)
        xm1 = 1.0 - x
        dcutdd = jnp.where(lt1, (-495.0 / CUTOFF) * x8 * xm1 * xm1, 0.0)
        s_rbfc[rowsn, 0:RES] = rbf
        s_rbfc[rowsn, RES:2 * RES] = drbf
        s_gsc[rowsn, 0:1] = 1.0 / d
        s_gsc[rowsn, 1:2] = invdx
        s_gsc[rowsn, 2:3] = cut
        s_gsc[rowsn, 3:4] = dcutdd
        s_gsc[rowsn, 4:7] = vec
        return tok

    def chunk_geom(cc):
        rowsn = pl.ds(cc * CHN, CHN)
        nm = nm_ref[0, rowsn, :]
        oh = s_ohc[rowsn, :]
        oht = s_ohtc[:, rowsn]
        rbf = s_rbfc[rowsn, 0:RES]
        drbf = s_rbfc[rowsn, RES:2 * RES]
        gsc = s_gsc[rowsn, :]
        invd = gsc[:, 0:1]
        invdx = gsc[:, 1:2]
        cut = gsc[:, 2:3]
        dcutdd = gsc[:, 3:4]
        vec = gsc[:, 4:7]
        v0, v1, v2 = gsc[:, 4:5], gsc[:, 5:6], gsc[:, 6:7]
        return (nm, oh, oht, vec, (v0, v1, v2), invd, invdx, rbf, drbf,
                cut, dcutdd)

    def chunk_edges(l, cc, geom):
        nm, oh, oht, vec, vs, invd, invdx, rbf, drbf, cut, dcutdd = geom
        lin = _dot(rbf, rbfw_ref[l]) + rbfb_ref[pl.ds(l, 1), :]
        rbf_m = lin * cut
        am_rows = s_am[pl.ds(cc * CH, CH), :]
        a_i = _dot1(rep, am_rows)
        a_g = _dot1(oh, s_am[...])
        msij = a_i * a_g * rbf_m
        fwrow = fw_ref[pl.ds(l, 1), :]
        s = jnp.sum(msij * fwrow, axis=1, keepdims=True)
        snm = s * nm
        return lin, rbf_m, a_i, a_g, msij, fwrow, s, snm

    def fwd_layer(l, carry):
        a, f0, f1, f2, r0, r1, r2 = carry
        s_a[l] = a
        s_r[3 * l] = r0
        s_r[3 * l + 1] = r1
        s_r[3 * l + 2] = r2
        s_f[3 * l] = f0
        s_f[3 * l + 1] = f1
        s_f[3 * l + 2] = f2
        s_am[...] = _mlp2(a, lw(10 * l), lb(8 * l), lw(10 * l + 1),
                          lb(8 * l + 1))

        def fwd_chunk(cc, tok):
            geom = chunk_geom(cc)
            nm, oh, oht, vec, (v0, v1, v2), invd, invdx = geom[:7]
            lin, rbf_m, a_i, a_g, msij, fwrow, s, snm = chunk_edges(l, cc,
                                                                    geom)
            fs = _mlp2(msij, lw(10 * l + 2), lb(8 * l + 2),
                       lw(10 * l + 3), lb(8 * l + 3))
            w = fs * snm
            pre = _mlp2(msij, lw(10 * l + 6), None,
                        lw(10 * l + 7), None) * nm
            rows = pl.ds(cc * CH, CH)
            for c3, vc in enumerate((v0, v1, v2)):
                vcn = vc * invdx
                s_t3[c3, rows, :] = _dot1(rept, w * vcn)
                rg = _dot1(oh, s_r[3 * l + c3])
                s_drx[c3, rows, :] = _dot1(rept, pre * rg)
            fdir_rows = _dot1(rept, vec * (snm * invdx))
            s_fdir[rows, :] = s_fdir[rows, :] + fdir_rows
            return tok

        jax.lax.fori_loop(0, C, fwd_chunk, 0)

        pr = _mlp2(a, lw(10 * l + 4), lb(8 * l + 4), lw(10 * l + 5),
                   lb(8 * l + 5))
        ea = _mlp2(a, lw(10 * l + 8), lb(8 * l + 6), lw(10 * l + 9),
                   lb(8 * l + 7))
        fi0, fi1, fi2 = s_t3[0], s_t3[1], s_t3[2]
        f0, f1, f2 = f0 + fi0, f1 + fi1, f2 + fi2
        r0 = r0 + pr * fi0 + s_drx[0]
        r1 = r1 + pr * fi1 + s_drx[1]
        r2 = r2 + pr * fi2 + s_drx[2]
        de = ea * -(f0 * r0 + f1 * r1 + f2 * r2)
        a = a + de
        return (a, f0, f1, f2, r0, r1, r2)

    jax.lax.fori_loop(0, C, geom_init, 0)
    zoh = (_iota((A, 10), 1) == zb).astype(_F32)
    a0 = _dot1(zoh, emb)
    zf = jnp.zeros((A, F), _F32)
    s_fdir[...] = jnp.zeros((A, 3), _F32)
    carry = jax.lax.fori_loop(
        0, NI, fwd_layer, (a0, zf, zf, zf, zf, zf, zf))
    a_fin, f0, f1, f2, r0, r1, r2 = carry
    s_a[NI] = a_fin
    s_r[3 * NI], s_r[3 * NI + 1], s_r[3 * NI + 2] = r0, r1, r2
    s_f[3 * NI], s_f[3 * NI + 1], s_f[3 * NI + 2] = f0, f1, f2

    at0, at0b = at0_ref[...], at0b_ref[...]
    at1, at1b = at1_ref[...], at1b_ref[...]
    at2, at2b = at2_ref[...], at2b_ref[...]
    amb = am_ref[0]
    z0h = _dot(a_fin, at0) + at0b
    h0 = _swish(z0h)
    z1h = _dot(h0, at1) + at1b
    h1 = _swish(z1h)
    ei = (jnp.sum(h1 * at2, axis=1, keepdims=True) + at2b) * amb
    e_ref[...] = jnp.reshape(jnp.sum(ei), (1, 1, 1))
    fd_ref[...] = s_fdir[...][None]

    d_h1 = amb * at2
    d_h0 = _dot(d_h1 * _swishp(z1h), at1t_ref[...])
    da_fin = _dot(d_h0 * _swishp(z0h), at0t_ref[...])

    s_dr[...] = jnp.zeros((A, 3), _F32)

    def bwd_layer(li, carry):
        l = NI - 1 - li
        da_out, dr0, dr1, dr2, df0, df1, df2 = carry
        a_in = s_a[l]
        s_am[...] = _mlp2(a_in, lw(10 * l), lb(8 * l), lw(10 * l + 1),
                          lb(8 * l + 1))
        pr = _mlp2(a_in, lw(10 * l + 4), lb(8 * l + 4), lw(10 * l + 5),
                   lb(8 * l + 5))
        z0e = _dot(a_in, lw(10 * l + 8)) + lb(8 * l + 6)
        ea = _dot(_swish(z0e), lw(10 * l + 9)) + lb(8 * l + 7)
        fo = (s_f[3 * l + 3], s_f[3 * l + 4], s_f[3 * l + 5])
        ro = (s_r[3 * l + 3], s_r[3 * l + 4], s_r[3 * l + 5])
        de0 = -(fo[0] * ro[0] + fo[1] * ro[1] + fo[2] * ro[2])
        d_ea = da_out * de0
        d_de0 = da_out * ea
        drp = (dr0 - d_de0 * fo[0], dr1 - d_de0 * fo[1],
               dr2 - d_de0 * fo[2])
        dfp = (df0 - d_de0 * ro[0], df1 - d_de0 * ro[1],
               df2 - d_de0 * ro[2])
        da = da_out + _dot(_dot(d_ea, lwt(10 * l + 9)) * _swishp(z0e),
                           lwt(10 * l + 8))
        for c3 in range(3):
            s_t3[c3] = drp[c3] * pr + dfp[c3]
            s_drx[c3] = drp[c3]
        s_dam[...] = jnp.zeros((A, F), _F32)
        s_dpr[...] = jnp.zeros((A, F), _F32)
        for c3 in range(3):
            s_drg[c3] = jnp.zeros((A, F), _F32)

        def bwd_chunk(cc, tok):
            geom = chunk_geom(cc)
            (nm, oh, oht, vec, (v0, v1, v2), invd, invdx, rbf, drbf,
             cut, dcutdd) = geom
            lin, rbf_m, a_i, a_g, msij, fwrow, s, snm = chunk_edges(l, cc,
                                                                    geom)
            z0fs = _dot(msij, lw(10 * l + 2)) + lb(8 * l + 2)
            fs = _dot(_swish(z0fs), lw(10 * l + 3)) + lb(8 * l + 3)
            z0re = _dot(msij, lw(10 * l + 6))
            pre = _dot(_swish(z0re), lw(10 * l + 7)) * nm
            w = fs * snm
            rows = pl.ds(cc * CH, CH)
            invdx2 = invdx * invdx
            dw = jnp.zeros((CHN, F), _F32)
            dvcs = []
            dpr_rows = jnp.zeros((CH, F), _F32)
            dd_geom = jnp.zeros((CHN, 1), _F32)
            for c3, vc in enumerate((v0, v1, v2)):
                vcn = vc * invdx
                dfi = s_t3[c3, rows, :]
                dfi_e = _dot1(rep, dfi)
                dw = dw + dfi_e * vcn
                dvc = jnp.sum(dfi_e * w, axis=1, keepdims=True)
                ddrx_rows = s_drx[c3, rows, :]
                dpr_rows = dpr_rows + ddrx_rows * _dot1(rept, w * vcn)
                ddrx_e = _dot1(rep, ddrx_rows)
                rg = _dot1(oh, s_r[3 * l + c3])
                dmsij_pre_part = ddrx_e * rg
                if c3 == 0:
                    dpre_raw = dmsij_pre_part
                else:
                    dpre_raw = dpre_raw + dmsij_pre_part
                s_drg[c3] = s_drg[c3] + _dot1(oht, ddrx_e * pre)
                dvn = dvc * invdx
                dd_geom = dd_geom - dvc * vc * invdx2
                dvcs.append(dvn)
            s_dpr[rows, :] = dpr_rows
            d_fs = dw * snm
            d_snm = jnp.sum(dw * fs, axis=1, keepdims=True)
            d_s = d_snm * nm
            dmsij = _mlp2_bwd(d_fs, msij, lw(10 * l + 2), lb(8 * l + 2),
                              lwt(10 * l + 2), lwt(10 * l + 3))
            dmsij = dmsij + _mlp2_bwd(dpre_raw * nm, msij,
                                      lw(10 * l + 6), None,
                                      lwt(10 * l + 6), lwt(10 * l + 7))
            dmsij = dmsij + d_s * fwrow
            d_ai = dmsij * a_g * rbf_m
            d_ag = dmsij * a_i * rbf_m
            d_rbfm = dmsij * a_i * a_g
            s_dam[rows, :] = s_dam[rows, :] + _dot1(rept, d_ai)
            s_dam[...] = s_dam[...] + _dot1(oht, d_ag)
            d_lin = d_rbfm * cut
            d_cut = jnp.sum(d_rbfm * lin, axis=1, keepdims=True)
            d_rbf = _dot(d_lin, rbfwt_ref[l])
            dd = jnp.sum(d_rbf * drbf, axis=1, keepdims=True)
            dd = dd + d_cut * dcutdd + dd_geom
            dvec = (jnp.concatenate(dvcs, axis=1)
                    + dd * invd * vec)
            s_dr[...] = s_dr[...] + _dot1(oht, dvec)
            s_dr[rows, :] = s_dr[rows, :] - _dot1(rept, dvec)
            return tok

        jax.lax.fori_loop(0, C, bwd_chunk, 0)

        da = da + _mlp2_bwd(s_dam[...], a_in, lw(10 * l), lb(8 * l),
                            lwt(10 * l), lwt(10 * l + 1))
        da = da + _mlp2_bwd(s_dpr[...], a_in, lw(10 * l + 4),
                            lb(8 * l + 4), lwt(10 * l + 4),
                            lwt(10 * l + 5))
        dr_in = (s_drx[0] + s_drg[0], s_drx[1] + s_drg[1],
                 s_drx[2] + s_drg[2])
        return (da, dr_in[0], dr_in[1], dr_in[2], dfp[0], dfp[1], dfp[2])

    jax.lax.fori_loop(0, NI, bwd_layer,
                      (da_fin, zf, zf, zf, zf, zf, zf))
    ff_ref[...] = (-s_dr[...])[None]


def _prep(params):
    wff, bff, rbfw, rbfb, fw = [], [], [], [], []
    for lp in params['layers']:
        wff += [lp['phi_a'][0]['W'].T, lp['phi_a'][1]['W'].T,
                lp['phi_f_scale'][0]['W'].T, lp['phi_f_scale'][1]['W'].T,
                lp['phi_r'][0]['W'].T, lp['phi_r'][1]['W'].T,
                lp['phi_r_ext'][0]['W'].T, lp['phi_r_ext'][1]['W'].T,
                lp['phi_e'][0]['W'].T, lp['phi_e'][1]['W'].T]
        bff += [lp['phi_a'][0]['b'], lp['phi_a'][1]['b'],
                lp['phi_f_scale'][0]['b'], lp['phi_f_scale'][1]['b'],
                lp['phi_r'][0]['b'], lp['phi_r'][1]['b'],
                lp['phi_e'][0]['b'], lp['phi_e'][1]['b']]
        rbfw.append(lp['phi_rbf']['W'].T)
        rbfb.append(lp['phi_rbf']['b'])
        fw.append(lp['phi_f']['W'][0])
    at = params['atomic']
    wff = jnp.stack(wff)
    rbfw = jnp.stack(rbfw)
    at0 = at[0]['W'].T
    at1 = at[1]['W'].T
    return (wff, jnp.transpose(wff, (0, 2, 1)), jnp.stack(bff),
            rbfw, jnp.transpose(rbfw, (0, 2, 1)), jnp.stack(rbfb),
            jnp.stack(fw), params['emb'],
            at0, at[0]['b'][None, :], at1, at[1]['b'][None, :],
            at[2]['W'], at[2]['b'][None, :],
            at0.T, at1.T)


def _specs():
    batch = lambda shape: pl.BlockSpec((1,) + shape,
                                       lambda i: (i,) + (0,) * len(shape))
    full = lambda shape: pl.BlockSpec(shape, lambda i: (0,) * len(shape))
    in_specs = [
        batch((A, 3)),
        batch((A, 1)),
        batch((A * NN, 1)),
        batch((C, 1, CHN)),
        batch((A, 1)),
        batch((A * NN, 1)),
        full((10 * NI, F, F)),
        full((10 * NI, F, F)),
        full((8 * NI, F)),
        full((NI, RES, F)),
        full((NI, F, RES)),
        full((NI, F)),
        full((NI, F)),
        full((10, F)),
        full((F, 128)), full((1, 128)),
        full((128, 64)), full((1, 64)),
        full((1, 64)), full((1, 1)),
        full((128, F)), full((64, 128)),
    ]
    out_specs = [
        pl.BlockSpec((1, 1, 1), lambda i: (i, 0, 0)),
        pl.BlockSpec((1, A, 3), lambda i: (i, 0, 0)),
        pl.BlockSpec((1, A, 3), lambda i: (i, 0, 0)),
    ]
    scratch = [
        pltpu.VMEM((NI + 1, A, F), _F32),
        pltpu.VMEM((3 * (NI + 1), A, F), _F32),
        pltpu.VMEM((3 * (NI + 1), A, F), _F32),
        pltpu.VMEM((A, F), _F32),
        pltpu.VMEM((3, A, F), _F32),
        pltpu.VMEM((3, A, F), _F32),
        pltpu.VMEM((A, F), _F32),
        pltpu.VMEM((3, A, F), _F32),
        pltpu.VMEM((A, F), _F32),
        pltpu.VMEM((A, 3), _F32),
        pltpu.VMEM((A, 3), _F32),
        pltpu.VMEM((A * NN, A), _BF16),
        pltpu.VMEM((A, A * NN), _BF16),
        pltpu.VMEM((A * NN, 2 * RES), _F32),
        pltpu.VMEM((A * NN, 7), _F32),
    ]
    return in_specs, out_specs, scratch


@jax.jit
def kernel(R, Z, N, AM, NM, params):
    b = R.shape[0]
    prepped = _prep(params)
    in_specs, out_specs, scratch = _specs()
    out_shape = [
        jax.ShapeDtypeStruct((b, 1, 1), _F32),
        jax.ShapeDtypeStruct((b, A, 3), _F32),
        jax.ShapeDtypeStruct((b, A, 3), _F32),
    ]
    n32 = N.astype(jnp.int32)
    e, ff, fdir = pl.pallas_call(
        _body,
        grid=(b,),
        in_specs=in_specs,
        out_specs=out_specs,
        out_shape=out_shape,
        scratch_shapes=scratch,
        compiler_params=pltpu.CompilerParams(
            dimension_semantics=("parallel",),
        ),
    )(R, Z[..., None].astype(jnp.int32),
      n32.reshape(b, A * NN, 1), n32.reshape(b, C, 1, CHN),
      AM[..., None], NM.reshape(b, A * NN, 1), *prepped)
    return (e.reshape(b, 1), ff, fdir)


# speedup vs baseline: 1.0634x; 1.0634x over previous
import jax
import jax.numpy as jnp
from jax.experimental import pallas as pl
from jax.experimental.pallas import tpu as pltpu

A, NN, F, RES, NI = 128, 48, 128, 20, 3
CH = 16
C = A // CH
CHN = CH * NN
CUTOFF = 5.0
EPS = 1e-8
_F32 = jnp.float32
_C9, _C10, _C11 = 55.0, 99.0, 45.0
_SQ2C = (2.0 / CUTOFF) ** 0.5


def _sig(x):
    return 1.0 / (1.0 + jnp.exp(-x))


def _swish(x):
    return x * _sig(x)


def _swishp(z):
    s = _sig(z)
    return s * (1.0 + z * (1.0 - s))


_BF16 = jnp.bfloat16


def _split(x):
    xh = x.astype(_BF16)
    xl = (x - xh.astype(_F32)).astype(_BF16)
    return xh, xl


def _rawdot(x, y):
    return jnp.dot(x, y, preferred_element_type=_F32)


def _dot(x, y):
    xh, xl = _split(x)
    yh, yl = _split(y)
    return _rawdot(xh, yh) + (_rawdot(xh, yl) + _rawdot(xl, yh))


def _dot1(oh, y):
    ohb = oh.astype(_BF16)
    yh, yl = _split(y)
    return _rawdot(ohb, yh) + _rawdot(ohb, yl)


def _dotw(x, w):
    xh, xl = _split(x)
    wh, wl = w
    return _rawdot(xh, wh) + (_rawdot(xh, wl) + _rawdot(xl, wh))


def _iota(shape, dim):
    return jax.lax.broadcasted_iota(jnp.int32, shape, dim)


def _mlp2(x, w0, b0, w1, b1):
    h = _dotw(x, w0)
    if b0 is not None:
        h = h + b0
    y = _dotw(_swish(h), w1)
    return y + b1 if b1 is not None else y


def _mlp2_bwd(dy, x, w0, b0, w0t, w1t):
    z0 = _dotw(x, w0)
    if b0 is not None:
        z0 = z0 + b0
    return _dotw(_dotw(dy, w1t) * _swishp(z0), w0t)


def _body(r_ref, z_ref, ncol_ref, nrow_ref, am_ref, nm_ref,
          wffh_ref, wffl_ref, wffth_ref, wfftl_ref, bff_ref,
          rbfwh_ref, rbfwl_ref, rbfwth_ref, rbfwtl_ref, rbfb_ref,
          fw_ref, emb_ref, at0_ref, at0b_ref, at1_ref, at1b_ref,
          at2_ref, at2b_ref, at0t_ref, at1t_ref,
          e_ref, ff_ref, fd_ref,
          s_a, s_r, s_f, s_am, s_t3, s_drx, s_dam, s_drg, s_dpr,
          s_fdir, s_dr, s_ohc, s_ohtc, s_rbfc, s_gsc):
    rfull = r_ref[0]
    zb = z_ref[0]
    emb = emb_ref[...]

    ai_col = _iota((CHN, 1), 0) // NN
    rep = (_iota((CHN, CH), 1) == ai_col).astype(_BF16)
    rept = (_iota((CH, CHN), 0) == (_iota((1, CHN), 1) // NN)).astype(_BF16)
    nv = ((_iota((1, RES), 1) + 1).astype(_F32) * (jnp.pi / CUTOFF))

    def lw(k):
        return (wffh_ref[k], wffl_ref[k])

    def lwt(k):
        return (wffth_ref[k], wfftl_ref[k])

    def lb(k):
        return bff_ref[pl.ds(k, 1), :]

    def geom_init(cc, tok):
        rowsn = pl.ds(cc * CHN, CHN)
        ncol = ncol_ref[0, rowsn, :]
        nrow = nrow_ref[0, cc]
        oh = (_iota((CHN, A), 1) == ncol).astype(_BF16)
        oht = (_iota((A, CHN), 0) == nrow).astype(_BF16)
        s_ohc[rowsn, :] = oh
        s_ohtc[:, rowsn] = oht
        rg = _dot1(oh, rfull)
        ri = _dot1(rep, r_ref[0, pl.ds(cc * CH, CH), :])
        vec = rg - ri
        v0, v1, v2 = vec[:, 0:1], vec[:, 1:2], vec[:, 2:3]
        d = jnp.sqrt(v0 * v0 + v1 * v1 + v2 * v2 + EPS)
        dx = d + EPS
        invdx = 1.0 / dx
        sn = jnp.sin(nv * d)
        cn = jnp.cos(nv * d)
        rbf = _SQ2C * sn * invdx
        drbf = _SQ2C * (nv * cn * dx - sn) * (invdx * invdx)
        x = d * (1.0 / CUTOFF)
        x2 = x * x
        x4 = x2 * x2
        x8 = x4 * x4
        lt1 = x < 1.0
        cut = jnp.where(lt1, 1.0 - x8 * x * (_C9 - _C10 * x + _C11 * x2),
                        0.0)
        xm1 = 1.0 - x
        dcutdd = jnp.where(lt1, (-495.0 / CUTOFF) * x8 * xm1 * xm1, 0.0)
        s_rbfc[rowsn, 0:RES] = rbf
        s_rbfc[rowsn, RES:2 * RES] = drbf
        s_gsc[rowsn, 0:1] = 1.0 / d
        s_gsc[rowsn, 1:2] = invdx
        s_gsc[rowsn, 2:3] = cut
        s_gsc[rowsn, 3:4] = dcutdd
        s_gsc[rowsn, 4:7] = vec
        return tok

    def chunk_geom(cc):
        rowsn = pl.ds(cc * CHN, CHN)
        nm = nm_ref[0, rowsn, :]
        oh = s_ohc[rowsn, :]
        oht = s_ohtc[:, rowsn]
        rbf = s_rbfc[rowsn, 0:RES]
        drbf = s_rbfc[rowsn, RES:2 * RES]
        gsc = s_gsc[rowsn, :]
        invd = gsc[:, 0:1]
        invdx = gsc[:, 1:2]
        cut = gsc[:, 2:3]
        dcutdd = gsc[:, 3:4]
        vec = gsc[:, 4:7]
        v0, v1, v2 = gsc[:, 4:5], gsc[:, 5:6], gsc[:, 6:7]
        return (nm, oh, oht, vec, (v0, v1, v2), invd, invdx, rbf, drbf,
                cut, dcutdd)

    def chunk_edges(l, cc, geom):
        nm, oh, oht, vec, vs, invd, invdx, rbf, drbf, cut, dcutdd = geom
        lin = _dotw(rbf, (rbfwh_ref[l], rbfwl_ref[l])) + rbfb_ref[pl.ds(l, 1), :]
        rbf_m = lin * cut
        am_rows = s_am[pl.ds(cc * CH, CH), :]
        a_i = _dot1(rep, am_rows)
        a_g = _dot1(oh, s_am[...])
        msij = a_i * a_g * rbf_m
        fwrow = fw_ref[pl.ds(l, 1), :]
        s = jnp.sum(msij * fwrow, axis=1, keepdims=True)
        snm = s * nm
        return lin, rbf_m, a_i, a_g, msij, fwrow, s, snm

    def fwd_layer(l, carry):
        a, f0, f1, f2, r0, r1, r2 = carry
        s_a[l] = a
        s_r[3 * l] = r0
        s_r[3 * l + 1] = r1
        s_r[3 * l + 2] = r2
        s_f[3 * l] = f0
        s_f[3 * l + 1] = f1
        s_f[3 * l + 2] = f2
        s_am[...] = _mlp2(a, lw(10 * l), lb(8 * l), lw(10 * l + 1),
                          lb(8 * l + 1))

        def fwd_chunk(cc, tok):
            geom = chunk_geom(cc)
            nm, oh, oht, vec, (v0, v1, v2), invd, invdx = geom[:7]
            lin, rbf_m, a_i, a_g, msij, fwrow, s, snm = chunk_edges(l, cc,
                                                                    geom)
            fs = _mlp2(msij, lw(10 * l + 2), lb(8 * l + 2),
                       lw(10 * l + 3), lb(8 * l + 3))
            w = fs * snm
            pre = _mlp2(msij, lw(10 * l + 6), None,
                        lw(10 * l + 7), None) * nm
            rows = pl.ds(cc * CH, CH)
            for c3, vc in enumerate((v0, v1, v2)):
                vcn = vc * invdx
                s_t3[c3, rows, :] = _dot1(rept, w * vcn)
                rg = _dot1(oh, s_r[3 * l + c3])
                s_drx[c3, rows, :] = _dot1(rept, pre * rg)
            fdir_rows = _dot1(rept, vec * (snm * invdx))
            s_fdir[rows, :] = s_fdir[rows, :] + fdir_rows
            return tok

        jax.lax.fori_loop(0, C, fwd_chunk, 0)

        pr = _mlp2(a, lw(10 * l + 4), lb(8 * l + 4), lw(10 * l + 5),
                   lb(8 * l + 5))
        ea = _mlp2(a, lw(10 * l + 8), lb(8 * l + 6), lw(10 * l + 9),
                   lb(8 * l + 7))
        fi0, fi1, fi2 = s_t3[0], s_t3[1], s_t3[2]
        f0, f1, f2 = f0 + fi0, f1 + fi1, f2 + fi2
        r0 = r0 + pr * fi0 + s_drx[0]
        r1 = r1 + pr * fi1 + s_drx[1]
        r2 = r2 + pr * fi2 + s_drx[2]
        de = ea * -(f0 * r0 + f1 * r1 + f2 * r2)
        a = a + de
        return (a, f0, f1, f2, r0, r1, r2)

    jax.lax.fori_loop(0, C, geom_init, 0)
    zoh = (_iota((A, 10), 1) == zb).astype(_F32)
    a0 = _dot1(zoh, emb)
    zf = jnp.zeros((A, F), _F32)
    s_fdir[...] = jnp.zeros((A, 3), _F32)
    carry = jax.lax.fori_loop(
        0, NI, fwd_layer, (a0, zf, zf, zf, zf, zf, zf))
    a_fin, f0, f1, f2, r0, r1, r2 = carry
    s_a[NI] = a_fin
    s_r[3 * NI], s_r[3 * NI + 1], s_r[3 * NI + 2] = r0, r1, r2
    s_f[3 * NI], s_f[3 * NI + 1], s_f[3 * NI + 2] = f0, f1, f2

    at0, at0b = at0_ref[...], at0b_ref[...]
    at1, at1b = at1_ref[...], at1b_ref[...]
    at2, at2b = at2_ref[...], at2b_ref[...]
    amb = am_ref[0]
    z0h = _dot(a_fin, at0) + at0b
    h0 = _swish(z0h)
    z1h = _dot(h0, at1) + at1b
    h1 = _swish(z1h)
    ei = (jnp.sum(h1 * at2, axis=1, keepdims=True) + at2b) * amb
    e_ref[...] = jnp.reshape(jnp.sum(ei), (1, 1, 1))
    fd_ref[...] = s_fdir[...][None]

    d_h1 = amb * at2
    d_h0 = _dot(d_h1 * _swishp(z1h), at1t_ref[...])
    da_fin = _dot(d_h0 * _swishp(z0h), at0t_ref[...])

    s_dr[...] = jnp.zeros((A, 3), _F32)

    def bwd_layer(li, carry):
        l = NI - 1 - li
        da_out, dr0, dr1, dr2, df0, df1, df2 = carry
        a_in = s_a[l]
        s_am[...] = _mlp2(a_in, lw(10 * l), lb(8 * l), lw(10 * l + 1),
                          lb(8 * l + 1))
        pr = _mlp2(a_in, lw(10 * l + 4), lb(8 * l + 4), lw(10 * l + 5),
                   lb(8 * l + 5))
        z0e = _dotw(a_in, lw(10 * l + 8)) + lb(8 * l + 6)
        ea = _dotw(_swish(z0e), lw(10 * l + 9)) + lb(8 * l + 7)
        fo = (s_f[3 * l + 3], s_f[3 * l + 4], s_f[3 * l + 5])
        ro = (s_r[3 * l + 3], s_r[3 * l + 4], s_r[3 * l + 5])
        de0 = -(fo[0] * ro[0] + fo[1] * ro[1] + fo[2] * ro[2])
        d_ea = da_out * de0
        d_de0 = da_out * ea
        drp = (dr0 - d_de0 * fo[0], dr1 - d_de0 * fo[1],
               dr2 - d_de0 * fo[2])
        dfp = (df0 - d_de0 * ro[0], df1 - d_de0 * ro[1],
               df2 - d_de0 * ro[2])
        da = da_out + _dotw(_dotw(d_ea, lwt(10 * l + 9)) * _swishp(z0e),
                           lwt(10 * l + 8))
        for c3 in range(3):
            s_t3[c3] = drp[c3] * pr + dfp[c3]
            s_drx[c3] = drp[c3]
        s_dam[...] = jnp.zeros((A, F), _F32)
        s_dpr[...] = jnp.zeros((A, F), _F32)
        for c3 in range(3):
            s_drg[c3] = jnp.zeros((A, F), _F32)

        def bwd_chunk(cc, tok):
            geom = chunk_geom(cc)
            (nm, oh, oht, vec, (v0, v1, v2), invd, invdx, rbf, drbf,
             cut, dcutdd) = geom
            lin, rbf_m, a_i, a_g, msij, fwrow, s, snm = chunk_edges(l, cc,
                                                                    geom)
            z0fs = _dotw(msij, lw(10 * l + 2)) + lb(8 * l + 2)
            fs = _dotw(_swish(z0fs), lw(10 * l + 3)) + lb(8 * l + 3)
            z0re = _dotw(msij, lw(10 * l + 6))
            pre = _dotw(_swish(z0re), lw(10 * l + 7)) * nm
            w = fs * snm
            rows = pl.ds(cc * CH, CH)
            invdx2 = invdx * invdx
            dw = jnp.zeros((CHN, F), _F32)
            dvcs = []
            dpr_rows = jnp.zeros((CH, F), _F32)
            dd_geom = jnp.zeros((CHN, 1), _F32)
            for c3, vc in enumerate((v0, v1, v2)):
                vcn = vc * invdx
                dfi = s_t3[c3, rows, :]
                dfi_e = _dot1(rep, dfi)
                dw = dw + dfi_e * vcn
                dvc = jnp.sum(dfi_e * w, axis=1, keepdims=True)
                ddrx_rows = s_drx[c3, rows, :]
                dpr_rows = dpr_rows + ddrx_rows * _dot1(rept, w * vcn)
                ddrx_e = _dot1(rep, ddrx_rows)
                rg = _dot1(oh, s_r[3 * l + c3])
                dmsij_pre_part = ddrx_e * rg
                if c3 == 0:
                    dpre_raw = dmsij_pre_part
                else:
                    dpre_raw = dpre_raw + dmsij_pre_part
                s_drg[c3] = s_drg[c3] + _dot1(oht, ddrx_e * pre)
                dvn = dvc * invdx
                dd_geom = dd_geom - dvc * vc * invdx2
                dvcs.append(dvn)
            s_dpr[rows, :] = dpr_rows
            d_fs = dw * snm
            d_snm = jnp.sum(dw * fs, axis=1, keepdims=True)
            d_s = d_snm * nm
            dmsij = _mlp2_bwd(d_fs, msij, lw(10 * l + 2), lb(8 * l + 2),
                              lwt(10 * l + 2), lwt(10 * l + 3))
            dmsij = dmsij + _mlp2_bwd(dpre_raw * nm, msij,
                                      lw(10 * l + 6), None,
                                      lwt(10 * l + 6), lwt(10 * l + 7))
            dmsij = dmsij + d_s * fwrow
            d_ai = dmsij * a_g * rbf_m
            d_ag = dmsij * a_i * rbf_m
            d_rbfm = dmsij * a_i * a_g
            s_dam[rows, :] = s_dam[rows, :] + _dot1(rept, d_ai)
            s_dam[...] = s_dam[...] + _dot1(oht, d_ag)
            d_lin = d_rbfm * cut
            d_cut = jnp.sum(d_rbfm * lin, axis=1, keepdims=True)
            d_rbf = _dotw(d_lin, (rbfwth_ref[l], rbfwtl_ref[l]))
            dd = jnp.sum(d_rbf * drbf, axis=1, keepdims=True)
            dd = dd + d_cut * dcutdd + dd_geom
            dvec = (jnp.concatenate(dvcs, axis=1)
                    + dd * invd * vec)
            s_dr[...] = s_dr[...] + _dot1(oht, dvec)
            s_dr[rows, :] = s_dr[rows, :] - _dot1(rept, dvec)
            return tok

        jax.lax.fori_loop(0, C, bwd_chunk, 0)

        da = da + _mlp2_bwd(s_dam[...], a_in, lw(10 * l), lb(8 * l),
                            lwt(10 * l), lwt(10 * l + 1))
        da = da + _mlp2_bwd(s_dpr[...], a_in, lw(10 * l + 4),
                            lb(8 * l + 4), lwt(10 * l + 4),
                            lwt(10 * l + 5))
        dr_in = (s_drx[0] + s_drg[0], s_drx[1] + s_drg[1],
                 s_drx[2] + s_drg[2])
        return (da, dr_in[0], dr_in[1], dr_in[2], dfp[0], dfp[1], dfp[2])

    jax.lax.fori_loop(0, NI, bwd_layer,
                      (da_fin, zf, zf, zf, zf, zf, zf))
    ff_ref[...] = (-s_dr[...])[None]


def _prep(params):
    wff, bff, rbfw, rbfb, fw = [], [], [], [], []
    for lp in params['layers']:
        wff += [lp['phi_a'][0]['W'].T, lp['phi_a'][1]['W'].T,
                lp['phi_f_scale'][0]['W'].T, lp['phi_f_scale'][1]['W'].T,
                lp['phi_r'][0]['W'].T, lp['phi_r'][1]['W'].T,
                lp['phi_r_ext'][0]['W'].T, lp['phi_r_ext'][1]['W'].T,
                lp['phi_e'][0]['W'].T, lp['phi_e'][1]['W'].T]
        bff += [lp['phi_a'][0]['b'], lp['phi_a'][1]['b'],
                lp['phi_f_scale'][0]['b'], lp['phi_f_scale'][1]['b'],
                lp['phi_r'][0]['b'], lp['phi_r'][1]['b'],
                lp['phi_e'][0]['b'], lp['phi_e'][1]['b']]
        rbfw.append(lp['phi_rbf']['W'].T)
        rbfb.append(lp['phi_rbf']['b'])
        fw.append(lp['phi_f']['W'][0])
    at = params['atomic']
    wff = jnp.stack(wff)
    wfft = jnp.transpose(wff, (0, 2, 1))
    rbfw = jnp.stack(rbfw)
    rbfwt = jnp.transpose(rbfw, (0, 2, 1))
    at0 = at[0]['W'].T
    at1 = at[1]['W'].T

    def sp(w):
        wh = w.astype(jnp.bfloat16)
        wl = (w - wh.astype(_F32)).astype(jnp.bfloat16)
        return wh, wl

    return (*sp(wff), *sp(wfft), jnp.stack(bff),
            *sp(rbfw), *sp(rbfwt), jnp.stack(rbfb),
            jnp.stack(fw), params['emb'],
            at0, at[0]['b'][None, :], at1, at[1]['b'][None, :],
            at[2]['W'], at[2]['b'][None, :],
            at0.T, at1.T)


def _specs():
    batch = lambda shape: pl.BlockSpec((1,) + shape,
                                       lambda i: (i,) + (0,) * len(shape))
    full = lambda shape: pl.BlockSpec(shape, lambda i: (0,) * len(shape))
    in_specs = [
        batch((A, 3)),
        batch((A, 1)),
        batch((A * NN, 1)),
        batch((C, 1, CHN)),
        batch((A, 1)),
        batch((A * NN, 1)),
        full((10 * NI, F, F)),
        full((10 * NI, F, F)),
        full((10 * NI, F, F)),
        full((10 * NI, F, F)),
        full((8 * NI, F)),
        full((NI, RES, F)),
        full((NI, RES, F)),
        full((NI, F, RES)),
        full((NI, F, RES)),
        full((NI, F)),
        full((NI, F)),
        full((10, F)),
        full((F, 128)), full((1, 128)),
        full((128, 64)), full((1, 64)),
        full((1, 64)), full((1, 1)),
        full((128, F)), full((64, 128)),
    ]
    out_specs = [
        pl.BlockSpec((1, 1, 1), lambda i: (i, 0, 0)),
        pl.BlockSpec((1, A, 3), lambda i: (i, 0, 0)),
        pl.BlockSpec((1, A, 3), lambda i: (i, 0, 0)),
    ]
    scratch = [
        pltpu.VMEM((NI + 1, A, F), _F32),
        pltpu.VMEM((3 * (NI + 1), A, F), _F32),
        pltpu.VMEM((3 * (NI + 1), A, F), _F32),
        pltpu.VMEM((A, F), _F32),
        pltpu.VMEM((3, A, F), _F32),
        pltpu.VMEM((3, A, F), _F32),
        pltpu.VMEM((A, F), _F32),
        pltpu.VMEM((3, A, F), _F32),
        pltpu.VMEM((A, F), _F32),
        pltpu.VMEM((A, 3), _F32),
        pltpu.VMEM((A, 3), _F32),
        pltpu.VMEM((A * NN, A), _BF16),
        pltpu.VMEM((A, A * NN), _BF16),
        pltpu.VMEM((A * NN, 2 * RES), _F32),
        pltpu.VMEM((A * NN, 7), _F32),
    ]
    return in_specs, out_specs, scratch


@jax.jit
def kernel(R, Z, N, AM, NM, params):
    b = R.shape[0]
    prepped = _prep(params)
    in_specs, out_specs, scratch = _specs()
    out_shape = [
        jax.ShapeDtypeStruct((b, 1, 1), _F32),
        jax.ShapeDtypeStruct((b, A, 3), _F32),
        jax.ShapeDtypeStruct((b, A, 3), _F32),
    ]
    n32 = N.astype(jnp.int32)
    e, ff, fdir = pl.pallas_call(
        _body,
        grid=(b,),
        in_specs=in_specs,
        out_specs=out_specs,
        out_shape=out_shape,
        scratch_shapes=scratch,
        compiler_params=pltpu.CompilerParams(
            dimension_semantics=("parallel",),
        ),
    )(R, Z[..., None].astype(jnp.int32),
      n32.reshape(b, A * NN, 1), n32.reshape(b, C, 1, CHN),
      AM[..., None], NM.reshape(b, A * NN, 1), *prepped)
    return (e.reshape(b, 1), ff, fdir)


# speedup vs baseline: 1.0664x; 1.0028x over previous
import jax
import jax.numpy as jnp
from jax.experimental import pallas as pl
from jax.experimental.pallas import tpu as pltpu

A, NN, F, RES, NI = 128, 48, 128, 20, 3
CH = 16
C = A // CH
CHN = CH * NN
CUTOFF = 5.0
EPS = 1e-8
_F32 = jnp.float32
_C9, _C10, _C11 = 55.0, 99.0, 45.0
_SQ2C = (2.0 / CUTOFF) ** 0.5


def _sig(x):
    return 1.0 / (1.0 + jnp.exp(-x))


def _swish(x):
    return x * _sig(x)


def _swishp(z):
    s = _sig(z)
    return s * (1.0 + z * (1.0 - s))


_BF16 = jnp.bfloat16


def _split(x):
    xh = x.astype(_BF16)
    xl = (x - xh.astype(_F32)).astype(_BF16)
    return xh, xl


def _rawdot(x, y):
    return jnp.dot(x, y, preferred_element_type=_F32)


def _dot(x, y):
    xh, xl = _split(x)
    yh, yl = _split(y)
    return _rawdot(xh, yh) + (_rawdot(xh, yl) + _rawdot(xl, yh))


def _dot1(oh, y):
    ohb = oh.astype(_BF16)
    yh, yl = _split(y)
    return _rawdot(ohb, yh) + _rawdot(ohb, yl)


def _dotw(x, w):
    xh, xl = _split(x)
    wh, wl = w
    return _rawdot(xh, wh) + (_rawdot(xh, wl) + _rawdot(xl, wh))


def _iota(shape, dim):
    return jax.lax.broadcasted_iota(jnp.int32, shape, dim)


def _mlp2(x, w0, b0, w1, b1):
    h = _dotw(x, w0)
    if b0 is not None:
        h = h + b0
    y = _dotw(_swish(h), w1)
    return y + b1 if b1 is not None else y


def _mlp2_bwd(dy, x, w0, b0, w0t, w1t):
    z0 = _dotw(x, w0)
    if b0 is not None:
        z0 = z0 + b0
    return _dotw(_dotw(dy, w1t) * _swishp(z0), w0t)


def _body(r_ref, z_ref, ncol_ref, nrow_ref, am_ref, nm_ref,
          wffh_ref, wffl_ref, wffth_ref, wfftl_ref, bff_ref,
          rbfwh_ref, rbfwl_ref, rbfwth_ref, rbfwtl_ref, rbfb_ref,
          fw_ref, emb_ref, at0_ref, at0b_ref, at1_ref, at1b_ref,
          at2_ref, at2b_ref, at0t_ref, at1t_ref,
          e_ref, ff_ref, fd_ref,
          s_a, s_r, s_f, s_am, s_t3, s_drx, s_dam, s_drg, s_dpr,
          s_fdir, s_dr, s_ohc, s_ohtc, s_rbfc, s_gsc):
    rfull = r_ref[0]
    zb = z_ref[0]
    emb = emb_ref[...]

    ai_col = _iota((CHN, 1), 0) // NN
    rep = (_iota((CHN, CH), 1) == ai_col).astype(_BF16)
    rept = (_iota((CH, CHN), 0) == (_iota((1, CHN), 1) // NN)).astype(_BF16)
    nv = ((_iota((1, RES), 1) + 1).astype(_F32) * (jnp.pi / CUTOFF))

    def lw(k):
        return (wffh_ref[k], wffl_ref[k])

    def lwt(k):
        return (wffth_ref[k], wfftl_ref[k])

    def lb(k):
        return bff_ref[pl.ds(k, 1), :]

    def geom_init(cc, tok):
        rowsn = pl.ds(cc * CHN, CHN)
        ncol = ncol_ref[0, rowsn, :]
        nrow = nrow_ref[0, cc]
        oh = (_iota((CHN, A), 1) == ncol).astype(_BF16)
        oht = (_iota((A, CHN), 0) == nrow).astype(_BF16)
        s_ohc[rowsn, :] = oh
        s_ohtc[cc] = oht
        rg = _dot1(oh, rfull)
        ri = _dot1(rep, r_ref[0, pl.ds(cc * CH, CH), :])
        vec = rg - ri
        v0, v1, v2 = vec[:, 0:1], vec[:, 1:2], vec[:, 2:3]
        d = jnp.sqrt(v0 * v0 + v1 * v1 + v2 * v2 + EPS)
        dx = d + EPS
        invdx = 1.0 / dx
        sn = jnp.sin(nv * d)
        cn = jnp.cos(nv * d)
        rbf = _SQ2C * sn * invdx
        drbf = _SQ2C * (nv * cn * dx - sn) * (invdx * invdx)
        x = d * (1.0 / CUTOFF)
        x2 = x * x
        x4 = x2 * x2
        x8 = x4 * x4
        lt1 = x < 1.0
        cut = jnp.where(lt1, 1.0 - x8 * x * (_C9 - _C10 * x + _C11 * x2),
                        0.0)
        xm1 = 1.0 - x
        dcutdd = jnp.where(lt1, (-495.0 / CUTOFF) * x8 * xm1 * xm1, 0.0)
        s_rbfc[rowsn, 0:RES] = rbf
        s_rbfc[rowsn, RES:2 * RES] = drbf
        s_gsc[rowsn, 0:1] = 1.0 / d
        s_gsc[rowsn, 1:2] = invdx
        s_gsc[rowsn, 2:3] = cut
        s_gsc[rowsn, 3:4] = dcutdd
        s_gsc[rowsn, 4:7] = vec
        return tok

    def chunk_geom(cc):
        rowsn = pl.ds(cc * CHN, CHN)
        nm = nm_ref[0, rowsn, :]
        oh = s_ohc[rowsn, :]
        oht = s_ohtc[cc]
        rbf = s_rbfc[rowsn, 0:RES]
        drbf = s_rbfc[rowsn, RES:2 * RES]
        gsc = s_gsc[rowsn, :]
        invd = gsc[:, 0:1]
        invdx = gsc[:, 1:2]
        cut = gsc[:, 2:3]
        dcutdd = gsc[:, 3:4]
        vec = gsc[:, 4:7]
        v0, v1, v2 = gsc[:, 4:5], gsc[:, 5:6], gsc[:, 6:7]
        return (nm, oh, oht, vec, (v0, v1, v2), invd, invdx, rbf, drbf,
                cut, dcutdd)

    def chunk_edges(l, cc, geom):
        nm, oh, oht, vec, vs, invd, invdx, rbf, drbf, cut, dcutdd = geom
        lin = _dotw(rbf, (rbfwh_ref[l], rbfwl_ref[l])) + rbfb_ref[pl.ds(l, 1), :]
        rbf_m = lin * cut
        am_rows = s_am[pl.ds(cc * CH, CH), :]
        a_i = _dot1(rep, am_rows)
        a_g = _dot1(oh, s_am[...])
        msij = a_i * a_g * rbf_m
        fwrow = fw_ref[pl.ds(l, 1), :]
        s = jnp.sum(msij * fwrow, axis=1, keepdims=True)
        snm = s * nm
        return lin, rbf_m, a_i, a_g, msij, fwrow, s, snm

    def fwd_layer(l, carry):
        a, f0, f1, f2, r0, r1, r2 = carry
        s_a[l] = a
        s_r[3 * l] = r0
        s_r[3 * l + 1] = r1
        s_r[3 * l + 2] = r2
        s_f[3 * l] = f0
        s_f[3 * l + 1] = f1
        s_f[3 * l + 2] = f2
        s_am[...] = _mlp2(a, lw(10 * l), lb(8 * l), lw(10 * l + 1),
                          lb(8 * l + 1))

        def fwd_chunk(cc, tok):
            geom = chunk_geom(cc)
            nm, oh, oht, vec, (v0, v1, v2), invd, invdx = geom[:7]
            lin, rbf_m, a_i, a_g, msij, fwrow, s, snm = chunk_edges(l, cc,
                                                                    geom)
            fs = _mlp2(msij, lw(10 * l + 2), lb(8 * l + 2),
                       lw(10 * l + 3), lb(8 * l + 3))
            w = fs * snm
            pre = _mlp2(msij, lw(10 * l + 6), None,
                        lw(10 * l + 7), None) * nm
            rows = pl.ds(cc * CH, CH)
            for c3, vc in enumerate((v0, v1, v2)):
                vcn = vc * invdx
                s_t3[c3, rows, :] = _dot1(rept, w * vcn)
                rg = _dot1(oh, s_r[3 * l + c3])
                s_drx[c3, rows, :] = _dot1(rept, pre * rg)
            fdir_rows = _dot1(rept, vec * (snm * invdx))
            s_fdir[rows, :] = s_fdir[rows, :] + fdir_rows
            return tok

        jax.lax.fori_loop(0, C, fwd_chunk, 0)

        pr = _mlp2(a, lw(10 * l + 4), lb(8 * l + 4), lw(10 * l + 5),
                   lb(8 * l + 5))
        ea = _mlp2(a, lw(10 * l + 8), lb(8 * l + 6), lw(10 * l + 9),
                   lb(8 * l + 7))
        fi0, fi1, fi2 = s_t3[0], s_t3[1], s_t3[2]
        f0, f1, f2 = f0 + fi0, f1 + fi1, f2 + fi2
        r0 = r0 + pr * fi0 + s_drx[0]
        r1 = r1 + pr * fi1 + s_drx[1]
        r2 = r2 + pr * fi2 + s_drx[2]
        de = ea * -(f0 * r0 + f1 * r1 + f2 * r2)
        a = a + de
        return (a, f0, f1, f2, r0, r1, r2)

    jax.lax.fori_loop(0, C, geom_init, 0)
    zoh = (_iota((A, 10), 1) == zb).astype(_F32)
    a0 = _dot1(zoh, emb)
    zf = jnp.zeros((A, F), _F32)
    s_fdir[...] = jnp.zeros((A, 3), _F32)
    carry = jax.lax.fori_loop(
        0, NI, fwd_layer, (a0, zf, zf, zf, zf, zf, zf))
    a_fin, f0, f1, f2, r0, r1, r2 = carry
    s_a[NI] = a_fin
    s_r[3 * NI], s_r[3 * NI + 1], s_r[3 * NI + 2] = r0, r1, r2
    s_f[3 * NI], s_f[3 * NI + 1], s_f[3 * NI + 2] = f0, f1, f2

    at0, at0b = at0_ref[...], at0b_ref[...]
    at1, at1b = at1_ref[...], at1b_ref[...]
    at2, at2b = at2_ref[...], at2b_ref[...]
    amb = am_ref[0]
    z0h = _dot(a_fin, at0) + at0b
    h0 = _swish(z0h)
    z1h = _dot(h0, at1) + at1b
    h1 = _swish(z1h)
    ei = (jnp.sum(h1 * at2, axis=1, keepdims=True) + at2b) * amb
    e_ref[...] = jnp.reshape(jnp.sum(ei), (1, 1, 1))
    fd_ref[...] = s_fdir[...][None]

    d_h1 = amb * at2
    d_h0 = _dot(d_h1 * _swishp(z1h), at1t_ref[...])
    da_fin = _dot(d_h0 * _swishp(z0h), at0t_ref[...])

    s_dr[...] = jnp.zeros((A, 3), _F32)

    def bwd_layer(li, carry):
        l = NI - 1 - li
        da_out, dr0, dr1, dr2, df0, df1, df2 = carry
        a_in = s_a[l]
        s_am[...] = _mlp2(a_in, lw(10 * l), lb(8 * l), lw(10 * l + 1),
                          lb(8 * l + 1))
        pr = _mlp2(a_in, lw(10 * l + 4), lb(8 * l + 4), lw(10 * l + 5),
                   lb(8 * l + 5))
        z0e = _dotw(a_in, lw(10 * l + 8)) + lb(8 * l + 6)
        ea = _dotw(_swish(z0e), lw(10 * l + 9)) + lb(8 * l + 7)
        fo = (s_f[3 * l + 3], s_f[3 * l + 4], s_f[3 * l + 5])
        ro = (s_r[3 * l + 3], s_r[3 * l + 4], s_r[3 * l + 5])
        de0 = -(fo[0] * ro[0] + fo[1] * ro[1] + fo[2] * ro[2])
        d_ea = da_out * de0
        d_de0 = da_out * ea
        drp = (dr0 - d_de0 * fo[0], dr1 - d_de0 * fo[1],
               dr2 - d_de0 * fo[2])
        dfp = (df0 - d_de0 * ro[0], df1 - d_de0 * ro[1],
               df2 - d_de0 * ro[2])
        da = da_out + _dotw(_dotw(d_ea, lwt(10 * l + 9)) * _swishp(z0e),
                           lwt(10 * l + 8))
        for c3 in range(3):
            s_t3[c3] = drp[c3] * pr + dfp[c3]
            s_drx[c3] = drp[c3]
        s_dam[...] = jnp.zeros((A, F), _F32)
        s_dpr[...] = jnp.zeros((A, F), _F32)
        for c3 in range(3):
            s_drg[c3] = jnp.zeros((A, F), _F32)

        def bwd_chunk(cc, tok):
            geom = chunk_geom(cc)
            (nm, oh, oht, vec, (v0, v1, v2), invd, invdx, rbf, drbf,
             cut, dcutdd) = geom
            lin, rbf_m, a_i, a_g, msij, fwrow, s, snm = chunk_edges(l, cc,
                                                                    geom)
            z0fs = _dotw(msij, lw(10 * l + 2)) + lb(8 * l + 2)
            fs = _dotw(_swish(z0fs), lw(10 * l + 3)) + lb(8 * l + 3)
            z0re = _dotw(msij, lw(10 * l + 6))
            pre = _dotw(_swish(z0re), lw(10 * l + 7)) * nm
            w = fs * snm
            rows = pl.ds(cc * CH, CH)
            invdx2 = invdx * invdx
            dw = jnp.zeros((CHN, F), _F32)
            dvcs = []
            dpr_rows = jnp.zeros((CH, F), _F32)
            dd_geom = jnp.zeros((CHN, 1), _F32)
            for c3, vc in enumerate((v0, v1, v2)):
                vcn = vc * invdx
                dfi = s_t3[c3, rows, :]
                dfi_e = _dot1(rep, dfi)
                dw = dw + dfi_e * vcn
                dvc = jnp.sum(dfi_e * w, axis=1, keepdims=True)
                ddrx_rows = s_drx[c3, rows, :]
                dpr_rows = dpr_rows + ddrx_rows * _dot1(rept, w * vcn)
                ddrx_e = _dot1(rep, ddrx_rows)
                rg = _dot1(oh, s_r[3 * l + c3])
                dmsij_pre_part = ddrx_e * rg
                if c3 == 0:
                    dpre_raw = dmsij_pre_part
                else:
                    dpre_raw = dpre_raw + dmsij_pre_part
                s_drg[c3] = s_drg[c3] + _dot1(oht, ddrx_e * pre)
                dvn = dvc * invdx
                dd_geom = dd_geom - dvc * vc * invdx2
                dvcs.append(dvn)
            s_dpr[rows, :] = dpr_rows
            d_fs = dw * snm
            d_snm = jnp.sum(dw * fs, axis=1, keepdims=True)
            d_s = d_snm * nm
            dmsij = _mlp2_bwd(d_fs, msij, lw(10 * l + 2), lb(8 * l + 2),
                              lwt(10 * l + 2), lwt(10 * l + 3))
            dmsij = dmsij + _mlp2_bwd(dpre_raw * nm, msij,
                                      lw(10 * l + 6), None,
                                      lwt(10 * l + 6), lwt(10 * l + 7))
            dmsij = dmsij + d_s * fwrow
            d_ai = dmsij * a_g * rbf_m
            d_ag = dmsij * a_i * rbf_m
            d_rbfm = dmsij * a_i * a_g
            s_dam[rows, :] = s_dam[rows, :] + _dot1(rept, d_ai)
            s_dam[...] = s_dam[...] + _dot1(oht, d_ag)
            d_lin = d_rbfm * cut
            d_cut = jnp.sum(d_rbfm * lin, axis=1, keepdims=True)
            d_rbf = _dotw(d_lin, (rbfwth_ref[l], rbfwtl_ref[l]))
            dd = jnp.sum(d_rbf * drbf, axis=1, keepdims=True)
            dd = dd + d_cut * dcutdd + dd_geom
            dvec = (jnp.concatenate(dvcs, axis=1)
                    + dd * invd * vec)
            s_dr[...] = s_dr[...] + _dot1(oht, dvec)
            s_dr[rows, :] = s_dr[rows, :] - _dot1(rept, dvec)
            return tok

        jax.lax.fori_loop(0, C, bwd_chunk, 0)

        da = da + _mlp2_bwd(s_dam[...], a_in, lw(10 * l), lb(8 * l),
                            lwt(10 * l), lwt(10 * l + 1))
        da = da + _mlp2_bwd(s_dpr[...], a_in, lw(10 * l + 4),
                            lb(8 * l + 4), lwt(10 * l + 4),
                            lwt(10 * l + 5))
        dr_in = (s_drx[0] + s_drg[0], s_drx[1] + s_drg[1],
                 s_drx[2] + s_drg[2])
        return (da, dr_in[0], dr_in[1], dr_in[2], dfp[0], dfp[1], dfp[2])

    jax.lax.fori_loop(0, NI, bwd_layer,
                      (da_fin, zf, zf, zf, zf, zf, zf))
    ff_ref[...] = (-s_dr[...])[None]


def _prep(params):
    wff, bff, rbfw, rbfb, fw = [], [], [], [], []
    for lp in params['layers']:
        wff += [lp['phi_a'][0]['W'].T, lp['phi_a'][1]['W'].T,
                lp['phi_f_scale'][0]['W'].T, lp['phi_f_scale'][1]['W'].T,
                lp['phi_r'][0]['W'].T, lp['phi_r'][1]['W'].T,
                lp['phi_r_ext'][0]['W'].T, lp['phi_r_ext'][1]['W'].T,
                lp['phi_e'][0]['W'].T, lp['phi_e'][1]['W'].T]
        bff += [lp['phi_a'][0]['b'], lp['phi_a'][1]['b'],
                lp['phi_f_scale'][0]['b'], lp['phi_f_scale'][1]['b'],
                lp['phi_r'][0]['b'], lp['phi_r'][1]['b'],
                lp['phi_e'][0]['b'], lp['phi_e'][1]['b']]
        rbfw.append(lp['phi_rbf']['W'].T)
        rbfb.append(lp['phi_rbf']['b'])
        fw.append(lp['phi_f']['W'][0])
    at = params['atomic']
    wff = jnp.stack(wff)
    wfft = jnp.transpose(wff, (0, 2, 1))
    rbfw = jnp.stack(rbfw)
    rbfwt = jnp.transpose(rbfw, (0, 2, 1))
    at0 = at[0]['W'].T
    at1 = at[1]['W'].T

    def sp(w):
        wh = w.astype(jnp.bfloat16)
        wl = (w - wh.astype(_F32)).astype(jnp.bfloat16)
        return wh, wl

    return (*sp(wff), *sp(wfft), jnp.stack(bff),
            *sp(rbfw), *sp(rbfwt), jnp.stack(rbfb),
            jnp.stack(fw), params['emb'],
            at0, at[0]['b'][None, :], at1, at[1]['b'][None, :],
            at[2]['W'], at[2]['b'][None, :],
            at0.T, at1.T)


def _specs():
    batch = lambda shape: pl.BlockSpec((1,) + shape,
                                       lambda i: (i,) + (0,) * len(shape))
    full = lambda shape: pl.BlockSpec(shape, lambda i: (0,) * len(shape))
    in_specs = [
        batch((A, 3)),
        batch((A, 1)),
        batch((A * NN, 1)),
        batch((C, 1, CHN)),
        batch((A, 1)),
        batch((A * NN, 1)),
        full((10 * NI, F, F)),
        full((10 * NI, F, F)),
        full((10 * NI, F, F)),
        full((10 * NI, F, F)),
        full((8 * NI, F)),
        full((NI, RES, F)),
        full((NI, RES, F)),
        full((NI, F, RES)),
        full((NI, F, RES)),
        full((NI, F)),
        full((NI, F)),
        full((10, F)),
        full((F, 128)), full((1, 128)),
        full((128, 64)), full((1, 64)),
        full((1, 64)), full((1, 1)),
        full((128, F)), full((64, 128)),
    ]
    out_specs = [
        pl.BlockSpec((1, 1, 1), lambda i: (i, 0, 0)),
        pl.BlockSpec((1, A, 3), lambda i: (i, 0, 0)),
        pl.BlockSpec((1, A, 3), lambda i: (i, 0, 0)),
    ]
    scratch = [
        pltpu.VMEM((NI + 1, A, F), _F32),
        pltpu.VMEM((3 * (NI + 1), A, F), _F32),
        pltpu.VMEM((3 * (NI + 1), A, F), _F32),
        pltpu.VMEM((A, F), _F32),
        pltpu.VMEM((3, A, F), _F32),
        pltpu.VMEM((3, A, F), _F32),
        pltpu.VMEM((A, F), _F32),
        pltpu.VMEM((3, A, F), _F32),
        pltpu.VMEM((A, F), _F32),
        pltpu.VMEM((A, 3), _F32),
        pltpu.VMEM((A, 3), _F32),
        pltpu.VMEM((A * NN, A), _BF16),
        pltpu.VMEM((C, A, CHN), _BF16),
        pltpu.VMEM((A * NN, 2 * RES), _F32),
        pltpu.VMEM((A * NN, 7), _F32),
    ]
    return in_specs, out_specs, scratch


@jax.jit
def kernel(R, Z, N, AM, NM, params):
    b = R.shape[0]
    prepped = _prep(params)
    in_specs, out_specs, scratch = _specs()
    out_shape = [
        jax.ShapeDtypeStruct((b, 1, 1), _F32),
        jax.ShapeDtypeStruct((b, A, 3), _F32),
        jax.ShapeDtypeStruct((b, A, 3), _F32),
    ]
    n32 = N.astype(jnp.int32)
    e, ff, fdir = pl.pallas_call(
        _body,
        grid=(b,),
        in_specs=in_specs,
        out_specs=out_specs,
        out_shape=out_shape,
        scratch_shapes=scratch,
        compiler_params=pltpu.CompilerParams(
            dimension_semantics=("parallel",),
        ),
    )(R, Z[..., None].astype(jnp.int32),
      n32.reshape(b, A * NN, 1), n32.reshape(b, C, 1, CHN),
      AM[..., None], NM.reshape(b, A * NN, 1), *prepped)
    return (e.reshape(b, 1), ff, fdir)
